# baseline (speedup 1.0000x reference)
import functools

import jax
import jax.numpy as jnp
import numpy as np
from jax.experimental import pallas as pl


_TWO_PI = 2.0 * np.pi
_DIR_OFFSET = 0.78539


def _body(bp_ref, rt_ref, an_ref, out_ref):
    a = bp_ref[6]
    b = rt_ref[6]
    c = an_ref[6]

    suv = jnp.sin(jnp.concatenate([a + b, a - b], axis=0))
    u = suv[:4]
    v = suv[4:]
    s1 = (u + v) * 0.5
    s2 = (u - v) * 0.5

    x = b + c - _DIR_OFFSET
    m = x - jnp.floor(x / _TWO_PI) * _TWO_PI
    d = jnp.clip(jnp.floor(m / np.pi), 0.0, 1.0)

    for i in range(4):
        out_ref[i] = jnp.concatenate(
            [bp_ref[0:6, i], s1[i:i + 1], rt_ref[0:6, i], s2[i:i + 1],
             1.0 - d[i:i + 1], d[i:i + 1]], axis=0)


@functools.partial(jax.jit, static_argnames=("block_lanes",))
def _run(bpP, rtP, anP, block_lanes):
    C, B, N = bpP.shape
    grid = (N // block_lanes,)
    in_spec = pl.BlockSpec((C, B, block_lanes), lambda i: (0, 0, i))
    out_spec = pl.BlockSpec((B, 16, block_lanes), lambda i: (0, 0, i))
    return pl.pallas_call(
        _body,
        grid=grid,
        in_specs=[in_spec, in_spec, in_spec],
        out_specs=out_spec,
        out_shape=jax.ShapeDtypeStruct((B, 16, N), bpP.dtype),
    )(bpP, rtP, anP)


def kernel(box_preds, reg_targets, anchors):
    bpP = jnp.transpose(box_preds, (2, 0, 1))
    rtP = jnp.transpose(reg_targets, (2, 0, 1))
    anP = jnp.transpose(anchors, (2, 0, 1))
    outP = _run(bpP, rtP, anP, 35200)
    return jnp.transpose(outP, (0, 2, 1))


# speedup vs baseline: 1.0041x; 1.0041x over previous
import functools

import jax
import jax.numpy as jnp
import numpy as np
from jax.experimental import pallas as pl


_TWO_PI = 2.0 * np.pi
_DIR_OFFSET = 0.78539


def _body(bp_ref, rt_ref, an_ref, out_ref):
    a = bp_ref[6]
    b = rt_ref[6]
    c = an_ref[6]
    B = a.shape[0]

    suv = jnp.sin(jnp.concatenate([a + b, a - b], axis=0))
    u = suv[:B]
    v = suv[B:]
    s1 = (u + v) * 0.5
    s2 = (u - v) * 0.5

    x = b + c - _DIR_OFFSET
    m = x - jnp.floor(x / _TWO_PI) * _TWO_PI
    d = jnp.clip(jnp.floor(m / np.pi), 0.0, 1.0)

    for i in range(B):
        out_ref[i] = jnp.concatenate(
            [bp_ref[0:6, i], s1[i:i + 1], rt_ref[0:6, i], s2[i:i + 1],
             1.0 - d[i:i + 1], d[i:i + 1]], axis=0)


@functools.partial(jax.jit, static_argnames=("block_lanes",))
def _run(bpP, rtP, anP, block_lanes):
    C, B, N = bpP.shape
    grid = (N // block_lanes,)
    in_spec = pl.BlockSpec((C, B, block_lanes), lambda i: (0, 0, i))
    out_spec = pl.BlockSpec((B, 16, block_lanes), lambda i: (0, 0, i))
    return pl.pallas_call(
        _body,
        grid=grid,
        in_specs=[in_spec, in_spec, in_spec],
        out_specs=out_spec,
        out_shape=jax.ShapeDtypeStruct((B, 16, N), bpP.dtype),
    )(bpP, rtP, anP)


def kernel(box_preds, reg_targets, anchors):
    bpP = jnp.transpose(box_preds, (2, 0, 1))
    rtP = jnp.transpose(reg_targets, (2, 0, 1))
    anP = jnp.transpose(anchors, (2, 0, 1))
    outP = _run(bpP, rtP, anP, 21120)
    return jnp.transpose(outP, (0, 2, 1))
